# baseline (speedup 1.0000x reference)
import functools

import jax
import jax.numpy as jnp
from jax import lax
from jax.experimental import pallas as pl
from jax.experimental.pallas import tpu as pltpu
from jax.experimental.pallas import tpu_sc as plsc

NUM_NODES = 1000000
EMBED_DIM = 64
BATCH = 16384

_NC = 2
_NS = 16
_NW = _NC * _NS
_B_PER_W = BATCH // _NW
_CHUNK = 128
_NCHUNK = _B_PER_W // _CHUNK


def _sc_gather(grp_hbm, rem_hbm, pairs_hbm, out_hbm,
               grp_v, rem_v, g_v, out_v, sem0, sem1):
    wid = lax.axis_index("s") * _NC + lax.axis_index("c")
    sems = (sem0, sem1)
    pltpu.sync_copy(grp_hbm.at[wid], grp_v)
    pltpu.sync_copy(rem_hbm.at[wid], rem_v)
    copies = [None, None]
    copies[0] = pltpu.async_copy(pairs_hbm.at[grp_v.at[0]], g_v.at[0], sems[0])
    for j in range(_NCHUNK):
        if j + 1 < _NCHUNK:
            b = (j + 1) % 2
            copies[b] = pltpu.async_copy(
                pairs_hbm.at[grp_v.at[j + 1]], g_v.at[b], sems[b])
        copies[j % 2].wait()
        for g in range(_CHUNK // 16):
            rem16 = rem_v[j, pl.ds(g * 16, 16)]
            for l in range(16):
                i = g * 16 + l
                base = rem16[l] * EMBED_DIM
                for k in range(EMBED_DIM // 16):
                    out_v[j % 2, i, pl.ds(k * 16, 16)] = (
                        g_v[j % 2, i, pl.ds(base + k * 16, 16)])
        pltpu.sync_copy(
            out_v.at[j % 2],
            out_hbm.at[pl.ds(wid * _B_PER_W + j * _CHUNK, _CHUNK)])


@jax.jit
def kernel(batch, emb_weight):
    idx = batch.astype(jnp.int32)
    grp = (idx >> 1).reshape(_NW, _NCHUNK, _CHUNK)
    rem = (idx & 1).reshape(_NW, _NCHUNK, _CHUNK)
    one = lax.optimization_barrier(jnp.float32(1.0))
    pairs = emb_weight.reshape(NUM_NODES // 2, 2 * EMBED_DIM) * one
    mesh = plsc.VectorSubcoreMesh(core_axis_name="c", subcore_axis_name="s")
    call = functools.partial(
        pl.kernel,
        mesh=mesh,
        out_type=jax.ShapeDtypeStruct((BATCH, EMBED_DIM), jnp.float32),
        scratch_types=[
            pltpu.VMEM((_NCHUNK, _CHUNK), jnp.int32),
            pltpu.VMEM((_NCHUNK, _CHUNK), jnp.int32),
            pltpu.VMEM((2, _CHUNK, 2 * EMBED_DIM), jnp.float32),
            pltpu.VMEM((2, _CHUNK, EMBED_DIM), jnp.float32),
            pltpu.SemaphoreType.DMA,
            pltpu.SemaphoreType.DMA,
        ],
    )(_sc_gather)
    return call(grp, rem, pairs)


# speedup vs baseline: 1.2526x; 1.2526x over previous
import functools

import jax
import jax.numpy as jnp
from jax import lax
from jax.experimental import pallas as pl
from jax.experimental.pallas import tpu as pltpu
from jax.experimental.pallas import tpu_sc as plsc

NUM_NODES = 1000000
EMBED_DIM = 64
BATCH = 16384

_NC = 2
_NS = 16
_NW = _NC * _NS
_CHUNK = 128
_B_PER_W = BATCH // _NW
_NCHUNK = _B_PER_W // _CHUNK


def _sc_gather(idx_hbm, table_hbm, out_hbm, idx_v, rows_v, sem):
    wid = lax.axis_index("s") * _NC + lax.axis_index("c")
    pltpu.sync_copy(idx_hbm.at[wid], idx_v)
    copies = []
    for j in range(_NCHUNK):
        copies.append(
            pltpu.async_copy(table_hbm.at[idx_v.at[j]],
                             rows_v.at[pl.ds(j * _CHUNK, _CHUNK)], sem))
    for c in copies:
        c.wait()
    pltpu.sync_copy(rows_v, out_hbm.at[pl.ds(wid * _B_PER_W, _B_PER_W)])


@jax.jit
def kernel(batch, emb_weight):
    idx = batch.astype(jnp.int32).reshape(_NW, _NCHUNK, _CHUNK)
    mesh = plsc.VectorSubcoreMesh(core_axis_name="c", subcore_axis_name="s")
    call = functools.partial(
        pl.kernel,
        mesh=mesh,
        out_type=jax.ShapeDtypeStruct((BATCH, EMBED_DIM), jnp.float32),
        scratch_types=[
            pltpu.VMEM((_NCHUNK, _CHUNK), jnp.int32),
            pltpu.VMEM((_B_PER_W, EMBED_DIM), jnp.float32),
            pltpu.SemaphoreType.DMA,
        ],
        compiler_params=pltpu.CompilerParams(use_tc_tiling_on_sc=False),
    )(_sc_gather)
    return call(idx, emb_weight)


# speedup vs baseline: 2.1431x; 1.7109x over previous
import functools

import jax
import jax.numpy as jnp
from jax import lax
from jax.experimental import pallas as pl
from jax.experimental.pallas import tpu as pltpu
from jax.experimental.pallas import tpu_sc as plsc

NUM_NODES = 1000000
EMBED_DIM = 64
BATCH = 16384

_NC = 2
_NS = 16
_NW = _NC * _NS
_B_PER_W = BATCH // _NW


def _sc_gather(idx_hbm, table_hbm, out_hbm, idx_v, rows_v, sem):
    wid = lax.axis_index("s") * _NC + lax.axis_index("c")
    pltpu.sync_copy(idx_hbm.at[wid], idx_v)
    for g in range(_B_PER_W // 16):
        idx16 = idx_v[pl.ds(g * 16, 16)]
        for l in range(16):
            r = idx16[l]
            pltpu.async_copy(table_hbm.at[r], rows_v.at[g * 16 + l], sem)
    pltpu.make_async_copy(
        out_hbm.at[pl.ds(wid * _B_PER_W, _B_PER_W)], rows_v, sem).wait()
    pltpu.sync_copy(rows_v, out_hbm.at[pl.ds(wid * _B_PER_W, _B_PER_W)])


@jax.jit
def kernel(batch, emb_weight):
    idx = batch.astype(jnp.int32).reshape(_NW, _B_PER_W)
    mesh = plsc.VectorSubcoreMesh(core_axis_name="c", subcore_axis_name="s")
    call = functools.partial(
        pl.kernel,
        mesh=mesh,
        out_type=jax.ShapeDtypeStruct((BATCH, EMBED_DIM), jnp.float32),
        scratch_types=[
            pltpu.VMEM((_B_PER_W,), jnp.int32),
            pltpu.VMEM((_B_PER_W, EMBED_DIM), jnp.float32),
            pltpu.SemaphoreType.DMA,
        ],
    )(_sc_gather)
    return call(idx, emb_weight)


# speedup vs baseline: 2.6630x; 1.2426x over previous
import functools

import jax
import jax.numpy as jnp
from jax import lax
from jax.experimental import pallas as pl
from jax.experimental.pallas import tpu as pltpu
from jax.experimental.pallas import tpu_sc as plsc

NUM_NODES = 1000000
EMBED_DIM = 64
BATCH = 16384

_NC = 2
_NS = 16
_NW = _NC * _NS
_B_PER_W = BATCH // _NW


def _sc_gather(idx_hbm, table_hbm, out_hbm, idx_v, rows_v, sem):
    wid = lax.axis_index("s") * _NC + lax.axis_index("c")
    pltpu.sync_copy(idx_hbm.at[wid], idx_v)
    for g in range(_B_PER_W // 16):
        idx16 = idx_v[pl.ds(g * 16, 16)]
        for l in range(16):
            r = idx16[l]
            pltpu.async_copy(table_hbm.at[r], rows_v.at[g * 16 + l], sem)
    pltpu.make_async_copy(
        out_hbm.at[pl.ds(wid * _B_PER_W, _B_PER_W)], rows_v, sem).wait()
    pltpu.sync_copy(rows_v, out_hbm.at[pl.ds(wid * _B_PER_W, _B_PER_W)])


_TBLK = 8192
_NBLK = (NUM_NODES + _TBLK - 1) // _TBLK


def _tc_transpose(t_ref, out_ref):
    out_ref[...] = t_ref[...].T


@jax.jit
def kernel(batch, emb_weight):
    idx = batch.astype(jnp.int32).reshape(_NW, _B_PER_W)
    t = emb_weight.T
    table_rm = pl.pallas_call(
        _tc_transpose,
        grid=(_NBLK,),
        in_specs=[pl.BlockSpec((EMBED_DIM, _TBLK), lambda i: (0, i))],
        out_specs=pl.BlockSpec((_TBLK, EMBED_DIM), lambda i: (i, 0)),
        out_shape=jax.ShapeDtypeStruct((NUM_NODES, EMBED_DIM), jnp.float32),
    )(t)
    mesh = plsc.VectorSubcoreMesh(core_axis_name="c", subcore_axis_name="s")
    call = functools.partial(
        pl.kernel,
        mesh=mesh,
        out_type=jax.ShapeDtypeStruct((BATCH, EMBED_DIM), jnp.float32),
        scratch_types=[
            pltpu.VMEM((_B_PER_W,), jnp.int32),
            pltpu.VMEM((_B_PER_W, EMBED_DIM), jnp.float32),
            pltpu.SemaphoreType.DMA,
        ],
    )(_sc_gather)
    return call(idx, table_rm)


# speedup vs baseline: 2.8394x; 1.0663x over previous
import functools

import jax
import jax.numpy as jnp
from jax import lax
from jax.experimental import pallas as pl
from jax.experimental.pallas import tpu as pltpu
from jax.experimental.pallas import tpu_sc as plsc

NUM_NODES = 1000000
EMBED_DIM = 64
BATCH = 16384

_NC = 2
_NS = 16
_NW = _NC * _NS
_B_PER_W = BATCH // _NW


def _sc_gather(idx_hbm, table_hbm, out_hbm, idx_v, rows_v, sem):
    wid = lax.axis_index("s") * _NC + lax.axis_index("c")
    pltpu.sync_copy(idx_hbm.at[wid], idx_v)
    for g in range(_B_PER_W // 16):
        idx16 = idx_v[pl.ds(g * 16, 16)]
        for l in range(16):
            r = idx16[l]
            pltpu.async_copy(table_hbm.at[r], rows_v.at[g * 16 + l], sem)
    pltpu.make_async_copy(
        out_hbm.at[pl.ds(wid * _B_PER_W, _B_PER_W)], rows_v, sem).wait()
    pltpu.sync_copy(rows_v, out_hbm.at[pl.ds(wid * _B_PER_W, _B_PER_W)])


_TBLK = 16384
_NBLK = (NUM_NODES + _TBLK - 1) // _TBLK


def _tc_transpose(t_ref, out_ref):
    out_ref[...] = t_ref[...].T


@jax.jit
def kernel(batch, emb_weight):
    idx = batch.astype(jnp.int32).reshape(_NW, _B_PER_W)
    t = emb_weight.T
    table_rm = pl.pallas_call(
        _tc_transpose,
        grid=(_NBLK,),
        in_specs=[pl.BlockSpec((EMBED_DIM, _TBLK), lambda i: (0, i))],
        out_specs=pl.BlockSpec((_TBLK, EMBED_DIM), lambda i: (i, 0)),
        out_shape=jax.ShapeDtypeStruct((NUM_NODES, EMBED_DIM), jnp.float32),
    )(t)
    mesh = plsc.VectorSubcoreMesh(core_axis_name="c", subcore_axis_name="s")
    call = functools.partial(
        pl.kernel,
        mesh=mesh,
        out_type=jax.ShapeDtypeStruct((BATCH, EMBED_DIM), jnp.float32),
        scratch_types=[
            pltpu.VMEM((_B_PER_W,), jnp.int32),
            pltpu.VMEM((_B_PER_W, EMBED_DIM), jnp.float32),
            pltpu.SemaphoreType.DMA,
        ],
    )(_sc_gather)
    return call(idx, table_rm)


# speedup vs baseline: 2.9057x; 1.0233x over previous
import functools

import jax
import jax.numpy as jnp
from jax import lax
from jax.experimental import pallas as pl
from jax.experimental.pallas import tpu as pltpu
from jax.experimental.pallas import tpu_sc as plsc

NUM_NODES = 1000000
EMBED_DIM = 64
BATCH = 16384

_NC = 2
_NS = 16
_NW = _NC * _NS
_B_PER_W = BATCH // _NW


def _sc_gather(idx_hbm, table_hbm, out_hbm, idx_v, rows_v, sem):
    wid = lax.axis_index("s") * _NC + lax.axis_index("c")
    pltpu.sync_copy(idx_hbm.at[wid], idx_v)
    for g in range(_B_PER_W // 16):
        idx16 = idx_v[pl.ds(g * 16, 16)]
        for l in range(16):
            r = idx16[l]
            pltpu.async_copy(table_hbm.at[r], rows_v.at[g * 16 + l], sem)
    pltpu.make_async_copy(
        out_hbm.at[pl.ds(wid * _B_PER_W, _B_PER_W)], rows_v, sem).wait()
    pltpu.sync_copy(rows_v, out_hbm.at[pl.ds(wid * _B_PER_W, _B_PER_W)])


_TBLK = 32768
_NBLK = (NUM_NODES + _TBLK - 1) // _TBLK


def _tc_transpose(t_ref, out_ref):
    out_ref[...] = t_ref[...].T


@jax.jit
def kernel(batch, emb_weight):
    idx = batch.astype(jnp.int32).reshape(_NW, _B_PER_W)
    t = emb_weight.T
    table_rm = pl.pallas_call(
        _tc_transpose,
        grid=(_NBLK,),
        in_specs=[pl.BlockSpec((EMBED_DIM, _TBLK), lambda i: (0, i))],
        out_specs=pl.BlockSpec((_TBLK, EMBED_DIM), lambda i: (i, 0)),
        out_shape=jax.ShapeDtypeStruct((NUM_NODES, EMBED_DIM), jnp.float32),
    )(t)
    mesh = plsc.VectorSubcoreMesh(core_axis_name="c", subcore_axis_name="s")
    call = functools.partial(
        pl.kernel,
        mesh=mesh,
        out_type=jax.ShapeDtypeStruct((BATCH, EMBED_DIM), jnp.float32),
        scratch_types=[
            pltpu.VMEM((_B_PER_W,), jnp.int32),
            pltpu.VMEM((_B_PER_W, EMBED_DIM), jnp.float32),
            pltpu.SemaphoreType.DMA,
        ],
    )(_sc_gather)
    return call(idx, table_rm)
